# baseline (speedup 1.0000x reference)
import math

import jax
import jax.numpy as jnp
from jax import lax
from jax.experimental import pallas as pl
from jax.experimental.pallas import tpu as pltpu
from jax.experimental.pallas import tpu_sc as plsc

D_MODEL = 64
VOCAB = 1000000
BATCH = 4096
SEQ = 200
SCALE = math.sqrt(D_MODEL)

NC = 2
NS = 16
NW = NC * NS

K1_CHUNK = 320
K1_NCHUNKS = VOCAB // K1_CHUNK
K1_ROUNDS = -(-K1_NCHUNKS // NW)

ROWS_PER_W = BATCH // NW
IDX_GROUP = 32
IDX_SHIFT = 5
SEQ_SPLITS = ((0, 128), (128, 72))
SEL_GROUPS = tuple((g, 0, 16) for g in range(0, 192, 16)) + ((184, 8, 8),)


def _k1_body(table_hbm, tab2_hbm, a_v, b_v, s_a0, s_a1, s_b0, s_b1):
    s_a = (s_a0, s_a1)
    s_b = (s_b0, s_b1)
    wid = lax.axis_index("s") * NC + lax.axis_index("c")

    def read_chunk(c, b):
        pltpu.async_copy(
            table_hbm.at[pl.ds(c * K1_CHUNK, K1_CHUNK)], a_v.at[b], s_a[b]
        )

    def wait_read(b):
        pltpu.make_async_copy(
            table_hbm.at[pl.ds(0, K1_CHUNK)], a_v.at[b], s_a[b]
        ).wait()

    def wait_store(b):
        pltpu.make_async_copy(
            b_v.at[b], tab2_hbm.at[pl.ds(0, K1_CHUNK // 2)], s_b[b]
        ).wait()

    @pl.when(wid < K1_NCHUNKS)
    def _prologue():
        read_chunk(wid, 0)

    def step(t, b):
        c = wid + t * NW

        @pl.when(c + NW < K1_NCHUNKS)
        def _fire_next():
            read_chunk(c + NW, 1 - b)

        @pl.when(c < K1_NCHUNKS)
        def _process():
            wait_read(b)

            @pl.when(t >= 2)
            def _drain_prev_store():
                wait_store(b)

            @plsc.parallel_loop(0, K1_CHUNK // 2, unroll=4)
            def _relayout(pr):
                for half in range(2):
                    for col in range(D_MODEL // 16):
                        b_v[b, pr, pl.ds(half * 64 + col * 16, 16)] = a_v[
                            b, 2 * pr + half, pl.ds(col * 16, 16)
                        ]

            pltpu.async_copy(
                b_v.at[b],
                tab2_hbm.at[pl.ds(c * (K1_CHUNK // 2), K1_CHUNK // 2)],
                s_b[b],
            )

    def pair(i, carry):
        step(i * 2, 0)
        step(i * 2 + 1, 1)
        return carry

    lax.fori_loop(0, (K1_ROUNDS + 1) // 2, pair, 0)
    @pl.when(wid < K1_NCHUNKS)
    def _drain0():
        wait_store(0)

    @pl.when(wid + NW < K1_NCHUNKS)
    def _drain1():
        wait_store(1)


def _k2_body(x_hbm, tab2_hbm, out_hbm, idx_v, pidx_v, pair_v, sel_v,
             s_g0, s_g1, s_o0, s_o1, s_x):
    s_g = (s_g0, s_g1)
    s_o = (s_o0, s_o1)
    wid = lax.axis_index("s") * NC + lax.axis_index("c")
    row_base = wid * ROWS_PER_W

    def fire_idx_group(g):
        pltpu.async_copy(
            x_hbm.at[pl.ds(row_base + g * IDX_GROUP, IDX_GROUP)],
            idx_v.at[g & 1],
            s_x,
        )

    def wait_idx_group():
        pltpu.make_async_copy(
            x_hbm.at[pl.ds(row_base, IDX_GROUP)], idx_v.at[0], s_x
        ).wait()

    def compute_pidx(c, b):
        gb = (c >> IDX_SHIFT) & 1
        r = c & (IDX_GROUP - 1)
        for off in range(0, SEQ - 16, 16):
            pidx_v[b, pl.ds(off, 16)] = idx_v[gb, r, pl.ds(off, 16)] >> 1
        pidx_v[b, pl.ds(SEQ - 16, 16)] = idx_v[gb, r, pl.ds(SEQ - 16, 16)] >> 1

    def fire_gathers(b):
        for off, n in SEQ_SPLITS:
            pltpu.async_copy(
                tab2_hbm.at[pidx_v.at[b, pl.ds(off, n)]],
                pair_v.at[b, pl.ds(off, n)],
                s_g[b],
            )

    def wait_gathers(b):
        for off, n in SEQ_SPLITS:
            pltpu.make_async_copy(
                tab2_hbm.at[pidx_v.at[b, pl.ds(off, n)]],
                pair_v.at[b, pl.ds(off, n)],
                s_g[b],
            ).wait()

    def wait_store(b):
        pltpu.make_async_copy(
            sel_v.at[b], out_hbm.at[row_base], s_o[b]
        ).wait()

    pltpu.sync_copy(x_hbm.at[pl.ds(row_base, IDX_GROUP)], idx_v.at[0])
    fire_idx_group(1)
    compute_pidx(0, 0)
    fire_gathers(0)

    def step(c, b):
        nb = 1 - b

        @pl.when(c + 1 < ROWS_PER_W)
        def _fire_next():
            @pl.when(((c + 1) & (IDX_GROUP - 1)) == 0)
            def _wait_idx():
                wait_idx_group()

            compute_pidx(c + 1, nb)

            @pl.when(c >= 1)
            def _drain_prev_store():
                wait_store(nb)

            fire_gathers(nb)

        wait_gathers(b)

        gb = (c >> IDX_SHIFT) & 1
        r = c & (IDX_GROUP - 1)

        def group(g0, lane0, n):
            offs = (idx_v[gb, r, pl.ds(g0, 16)] & 1) * D_MODEL
            for j in range(n):
                off = pl.multiple_of(offs[lane0 + j], D_MODEL)
                s = g0 + lane0 + j
                for col in range(D_MODEL // 16):
                    sel_v[b, s, pl.ds(col * 16, 16)] = (
                        pair_v[b, s, pl.ds(off + col * 16, 16)] * SCALE
                    )

        for g0, lane0, n in SEL_GROUPS:
            group(g0, lane0, n)

        pltpu.async_copy(sel_v.at[b], out_hbm.at[row_base + c], s_o[b])

        @pl.when(((c & (IDX_GROUP - 1)) == (IDX_GROUP - 1))
                 & (c + 1 + IDX_GROUP < ROWS_PER_W))
        def _prefetch_idx():
            fire_idx_group((c >> IDX_SHIFT) + 2)

    def pair(i, carry):
        step(i * 2, 0)
        step(i * 2 + 1, 1)
        return carry

    lax.fori_loop(0, ROWS_PER_W // 2, pair, 0)
    wait_store(0)
    wait_store(1)


@jax.jit
def _embed(x, lut_weight):
    mesh = plsc.VectorSubcoreMesh(core_axis_name="c", subcore_axis_name="s")
    tab2 = pl.kernel(
        _k1_body,
        out_type=jax.ShapeDtypeStruct((VOCAB // 2, 2 * D_MODEL), jnp.float32),
        mesh=mesh,
        scratch_types=[
            pltpu.VMEM((2, K1_CHUNK, D_MODEL), jnp.float32),
            pltpu.VMEM((2, K1_CHUNK // 2, 2 * D_MODEL), jnp.float32),
            pltpu.SemaphoreType.DMA,
            pltpu.SemaphoreType.DMA,
            pltpu.SemaphoreType.DMA,
            pltpu.SemaphoreType.DMA,
        ],
        compiler_params=pltpu.CompilerParams(use_tc_tiling_on_sc=True),
    )(lut_weight)
    out = pl.kernel(
        _k2_body,
        out_type=jax.ShapeDtypeStruct((BATCH, SEQ, D_MODEL), jnp.float32),
        mesh=mesh,
        scratch_types=[
            pltpu.VMEM((2, IDX_GROUP, SEQ), jnp.int32),
            pltpu.VMEM((2, SEQ), jnp.int32),
            pltpu.VMEM((2, SEQ, 2 * D_MODEL), jnp.float32),
            pltpu.VMEM((2, SEQ, D_MODEL), jnp.float32),
            pltpu.SemaphoreType.DMA,
            pltpu.SemaphoreType.DMA,
            pltpu.SemaphoreType.DMA,
            pltpu.SemaphoreType.DMA,
            pltpu.SemaphoreType.DMA,
        ],
        compiler_params=pltpu.CompilerParams(use_tc_tiling_on_sc=True),
    )(x, tab2)
    return out


def kernel(x, lut_weight):
    return _embed(x, lut_weight)


# speedup vs baseline: 1.1189x; 1.1189x over previous
import math

import jax
import jax.numpy as jnp
from jax import lax
from jax.experimental import pallas as pl
from jax.experimental.pallas import tpu as pltpu
from jax.experimental.pallas import tpu_sc as plsc

D_MODEL = 64
VOCAB = 1000000
BATCH = 4096
SEQ = 200
SCALE = math.sqrt(D_MODEL)

NC = 2
NS = 16
NW = NC * NS

BLK = 256
NBLK = BATCH // BLK
NITEMS = SEQ * NBLK
ITEMS_PER_W = NITEMS // NW


def _body(xT_hbm, table_hbm, outT_hbm, idx_v, g_v, t_v,
          s_i0, s_i1, s_g0, s_g1, s_o0, s_o1):
    s_i = (s_i0, s_i1)
    s_g = (s_g0, s_g1)
    s_o = (s_o0, s_o1)
    wid = lax.axis_index("s") * NC + lax.axis_index("c")

    def item_sb(k):
        t = wid + k * NW
        return t >> 4, t & (NBLK - 1)

    def fire_idx(k, b):
        s, blk = item_sb(k)
        pltpu.async_copy(
            xT_hbm.at[s, pl.ds(blk * BLK, BLK)], idx_v.at[b], s_i[b]
        )

    def wait_idx(b):
        pltpu.make_async_copy(
            xT_hbm.at[0, pl.ds(0, BLK)], idx_v.at[b], s_i[b]
        ).wait()

    def fire_gathers(b):
        for off in range(0, BLK, 128):
            pltpu.async_copy(
                table_hbm.at[idx_v.at[b, pl.ds(off, 128)]],
                g_v.at[b, pl.ds(off, 128)],
                s_g[b],
            )

    def wait_gathers(b):
        for off in range(0, BLK, 128):
            pltpu.make_async_copy(
                table_hbm.at[idx_v.at[b, pl.ds(off, 128)]],
                g_v.at[b, pl.ds(off, 128)],
                s_g[b],
            ).wait()

    def fire_store(k, b):
        s, blk = item_sb(k)
        pltpu.async_copy(
            t_v.at[b], outT_hbm.at[s, :, pl.ds(blk * BLK, BLK)], s_o[b]
        )

    def wait_store(b):
        pltpu.make_async_copy(
            t_v.at[b], outT_hbm.at[0, :, pl.ds(0, BLK)], s_o[b]
        ).wait()

    s0, blk0 = item_sb(0)
    pltpu.sync_copy(xT_hbm.at[s0, pl.ds(blk0 * BLK, BLK)], idx_v.at[0])
    fire_gathers(0)
    fire_idx(1, 1)

    jvecs = [jnp.arange(g * 16, g * 16 + 16, dtype=jnp.int32)
             for g in range(BLK // 16)]

    def step(k, b):
        nb = 1 - b

        @pl.when(k + 1 < ITEMS_PER_W)
        def _fire_next():
            wait_idx(nb)

            @pl.when(k >= 1)
            def _drain_prev_store():
                wait_store(nb)

            fire_gathers(nb)

        wait_gathers(b)

        @pl.when(k + 2 < ITEMS_PER_W)
        def _prefetch_idx():
            fire_idx(k + 2, b)

        @plsc.parallel_loop(0, D_MODEL, unroll=2)
        def _transpose(d):
            dvec = jnp.full((16,), 0, jnp.int32) + d
            for g in range(BLK // 16):
                vals = plsc.load_gather(g_v.at[b], [jvecs[g], dvec])
                t_v[b, d, pl.ds(g * 16, 16)] = vals * SCALE

        fire_store(k, b)

    def pair(i, carry):
        step(i * 2, 0)
        step(i * 2 + 1, 1)
        return carry

    lax.fori_loop(0, ITEMS_PER_W // 2, pair, 0)
    wait_store(0)
    wait_store(1)


@jax.jit
def _embed(xT, lut_weight):
    mesh = plsc.VectorSubcoreMesh(core_axis_name="c", subcore_axis_name="s")
    outT = pl.kernel(
        _body,
        out_type=jax.ShapeDtypeStruct((SEQ, D_MODEL, BATCH), jnp.float32),
        mesh=mesh,
        scratch_types=[
            pltpu.VMEM((2, BLK), jnp.int32),
            pltpu.VMEM((2, BLK, D_MODEL), jnp.float32),
            pltpu.VMEM((2, D_MODEL, BLK), jnp.float32),
            pltpu.SemaphoreType.DMA,
            pltpu.SemaphoreType.DMA,
            pltpu.SemaphoreType.DMA,
            pltpu.SemaphoreType.DMA,
            pltpu.SemaphoreType.DMA,
            pltpu.SemaphoreType.DMA,
        ],
        compiler_params=pltpu.CompilerParams(
            use_tc_tiling_on_sc=False, needs_layout_passes=False
        ),
    )(xT, lut_weight)
    return outT


def kernel(x, lut_weight):
    xT = x.T
    outT = _embed(xT, lut_weight)
    return outT.transpose(2, 0, 1)


# speedup vs baseline: 1.2216x; 1.0918x over previous
import math

import jax
import jax.numpy as jnp
from jax import lax
from jax.experimental import pallas as pl
from jax.experimental.pallas import tpu as pltpu
from jax.experimental.pallas import tpu_sc as plsc

D_MODEL = 64
VOCAB = 1000000
BATCH = 4096
SEQ = 200
SCALE = math.sqrt(D_MODEL)

NC = 2
NS = 16
NW = NC * NS

BLK = 256
NBLK = BATCH // BLK
NITEMS = SEQ * NBLK
ITEMS_PER_W = NITEMS // NW


def _body(xT_hbm, tab2_hbm, outT_hbm, idx_v, pidx_v, h_v, g_v, t_v,
          s_i0, s_i1, s_g0, s_g1, s_o0, s_o1):
    s_i = (s_i0, s_i1)
    s_g = (s_g0, s_g1)
    s_o = (s_o0, s_o1)
    wid = lax.axis_index("s") * NC + lax.axis_index("c")

    def item_sb(k):
        t = wid + k * NW
        return t >> 4, t & (NBLK - 1)

    def fire_idx(k, b):
        s, blk = item_sb(k)
        pltpu.async_copy(
            xT_hbm.at[s, pl.ds(blk * BLK, BLK)], idx_v.at[b], s_i[b]
        )

    def wait_idx(b):
        pltpu.make_async_copy(
            xT_hbm.at[0, pl.ds(0, BLK)], idx_v.at[b], s_i[b]
        ).wait()

    def compute_ph(b):
        for g in range(BLK // 16):
            v = idx_v[b, pl.ds(g * 16, 16)]
            pidx_v[b, pl.ds(g * 16, 16)] = v >> 1
            h_v[b, pl.ds(g * 16, 16)] = (v & 1) * D_MODEL

    def fire_gathers(b):
        for off in range(0, BLK, 128):
            pltpu.async_copy(
                tab2_hbm.at[pidx_v.at[b, pl.ds(off, 128)]],
                g_v.at[b, pl.ds(off, 128)],
                s_g[b],
            )

    def wait_gathers(b):
        for off in range(0, BLK, 128):
            pltpu.make_async_copy(
                tab2_hbm.at[pidx_v.at[b, pl.ds(off, 128)]],
                g_v.at[b, pl.ds(off, 128)],
                s_g[b],
            ).wait()

    def fire_store(k, b):
        s, blk = item_sb(k)
        pltpu.async_copy(
            t_v.at[b], outT_hbm.at[s, :, pl.ds(blk * BLK, BLK)], s_o[b]
        )

    def wait_store(b):
        pltpu.make_async_copy(
            t_v.at[b], outT_hbm.at[0, :, pl.ds(0, BLK)], s_o[b]
        ).wait()

    s0, blk0 = item_sb(0)
    pltpu.sync_copy(xT_hbm.at[s0, pl.ds(blk0 * BLK, BLK)], idx_v.at[0])
    compute_ph(0)
    fire_gathers(0)
    fire_idx(1, 1)

    jvecs = [jnp.arange(g * 16, g * 16 + 16, dtype=jnp.int32)
             for g in range(BLK // 16)]

    def step(k, b):
        nb = 1 - b

        @pl.when(k + 1 < ITEMS_PER_W)
        def _fire_next():
            wait_idx(nb)
            compute_ph(nb)

            @pl.when(k >= 1)
            def _drain_prev_store():
                wait_store(nb)

            fire_gathers(nb)

        wait_gathers(b)

        @pl.when(k + 2 < ITEMS_PER_W)
        def _prefetch_idx():
            fire_idx(k + 2, b)

        for g in range(BLK // 16):
            h64 = h_v[b, pl.ds(g * 16, 16)]

            @plsc.parallel_loop(0, D_MODEL, unroll=2)
            def _transpose(d, _jv=jvecs[g], _h=h64, _g=g):
                vals = plsc.load_gather(g_v.at[b], [_jv, _h + d])
                t_v[b, d, pl.ds(_g * 16, 16)] = vals * SCALE

        fire_store(k, b)

    def pair(i, carry):
        step(i * 2, 0)
        step(i * 2 + 1, 1)
        return carry

    lax.fori_loop(0, ITEMS_PER_W // 2, pair, 0)
    wait_store(0)
    wait_store(1)


@jax.jit
def _embed(xT, tab2):
    mesh = plsc.VectorSubcoreMesh(core_axis_name="c", subcore_axis_name="s")
    outT = pl.kernel(
        _body,
        out_type=jax.ShapeDtypeStruct((SEQ, D_MODEL, BATCH), jnp.float32),
        mesh=mesh,
        scratch_types=[
            pltpu.VMEM((2, BLK), jnp.int32),
            pltpu.VMEM((2, BLK), jnp.int32),
            pltpu.VMEM((2, BLK), jnp.int32),
            pltpu.VMEM((2, BLK, 2 * D_MODEL), jnp.float32),
            pltpu.VMEM((2, D_MODEL, BLK), jnp.float32),
            pltpu.SemaphoreType.DMA,
            pltpu.SemaphoreType.DMA,
            pltpu.SemaphoreType.DMA,
            pltpu.SemaphoreType.DMA,
            pltpu.SemaphoreType.DMA,
            pltpu.SemaphoreType.DMA,
        ],
        compiler_params=pltpu.CompilerParams(
            use_tc_tiling_on_sc=True, needs_layout_passes=False
        ),
    )(xT, tab2)
    return outT


def kernel(x, lut_weight):
    xT = x.T
    tab2 = lut_weight.reshape(VOCAB // 2, 2 * D_MODEL)
    outT = _embed(xT, tab2)
    return outT.transpose(2, 0, 1)
